# baseline (speedup 1.0000x reference)
import functools

import jax
import jax.numpy as jnp
from jax import lax
from jax.experimental import pallas as pl
from jax.experimental.pallas import tpu as pltpu
from jax.experimental.pallas import tpu_sc as plsc

_B = 16
_N = 8192
_S = 2048
_HALF = _S // 2


_W = 256
_NB = _N // _W
_F = _W // 128


def _fold(t, op):
    acc = t[:, 0:128]
    for g in range(1, _F):
        acc = op(acc, t[:, g * 128:(g + 1) * 128])
    return acc


def _fps_tc_body(x_ref, y_ref, z_ref, idx_ref, dist_ref):
    r = jax.lax.broadcasted_iota(jnp.int32, (_B, _B), 0)
    c = jax.lax.broadcasted_iota(jnp.int32, (_B, _B), 1)
    eye = r == c
    dist_ref[...] = jnp.full((_B, _N), 1e10, jnp.float32)
    lane0 = jax.lax.broadcasted_iota(jnp.int32, (_B, _W), 1)

    def t_row(v):
        vb = jnp.broadcast_to(v, (_B, _B))
        return jnp.sum(jnp.where(eye, vb, jnp.zeros_like(vb)), axis=0,
                       keepdims=True)

    def body(i, far):
        idx_ref[pl.ds(i, 1), :] = t_row(far)

        ax = jnp.zeros((_B, 128), jnp.float32)
        ay = jnp.zeros((_B, 128), jnp.float32)
        az = jnp.zeros((_B, 128), jnp.float32)
        for blk in range(_NB):
            sl = pl.ds(blk * _W, _W)
            m2 = (lane0 + blk * _W) == far
            ax = ax + _fold(jnp.where(m2, x_ref[:, sl], 0.0), jnp.add)
            ay = ay + _fold(jnp.where(m2, y_ref[:, sl], 0.0), jnp.add)
            az = az + _fold(jnp.where(m2, z_ref[:, sl], 0.0), jnp.add)
        cx = jnp.sum(ax, axis=1, keepdims=True)
        cy = jnp.sum(ay, axis=1, keepdims=True)
        cz = jnp.sum(az, axis=1, keepdims=True)

        am = jnp.full((_B, 128), -1.0, jnp.float32)
        for blk in range(_NB):
            sl = pl.ds(blk * _W, _W)
            dx = x_ref[:, sl] - cx
            dy = y_ref[:, sl] - cy
            dz = z_ref[:, sl] - cz
            d = (dx * dx + dz * dz) + dy * dy
            nd = jnp.minimum(dist_ref[:, sl], d)
            dist_ref[:, sl] = nd
            am = jnp.maximum(am, _fold(nd, jnp.maximum))
        mx = jnp.max(am, axis=1, keepdims=True)

        ai = jnp.full((_B, 128), _N, jnp.int32)
        for blk in range(_NB):
            sl = pl.ds(blk * _W, _W)
            cand = jnp.where(dist_ref[:, sl] == mx, lane0 + blk * _W, _N)
            ai = jnp.minimum(ai, _fold(cand, jnp.minimum))
        far_new = jnp.min(ai, axis=1, keepdims=True)
        return far_new

    jax.lax.fori_loop(0, _S, body, jnp.zeros((_B, 1), jnp.int32))


def _fps_tc(x, y, z):
    return pl.pallas_call(
        _fps_tc_body,
        out_shape=jax.ShapeDtypeStruct((_S, _B), jnp.int32),
        scratch_shapes=[pltpu.VMEM((_B, _N), jnp.float32)],
    )(x, y, z)


_L = 16


def _gather_sc_body(xt, yt, zt, idxT, xo_out, yo_out, zo_out,
                    xs, ys, zs, idx_v, ox, oy, oz, sem):
    c = lax.axis_index("c")
    s = lax.axis_index("s")
    col = c * _HALF
    pltpu.sync_copy(xt.at[s], xs)
    pltpu.sync_copy(yt.at[s], ys)
    pltpu.sync_copy(zt.at[s], zs)
    pltpu.sync_copy(idxT.at[s, pl.ds(col, _HALF)], idx_v)

    @plsc.parallel_loop(0, _HALF, step=_L, unroll=8)
    def chunk(k):
        idxc = idx_v[pl.ds(k, _L)]
        ox[pl.ds(k, _L)] = plsc.load_gather(xs, [idxc])
        oy[pl.ds(k, _L)] = plsc.load_gather(ys, [idxc])
        oz[pl.ds(k, _L)] = plsc.load_gather(zs, [idxc])

    pltpu.sync_copy(ox, xo_out.at[s, pl.ds(col, _HALF)])
    pltpu.sync_copy(oy, yo_out.at[s, pl.ds(col, _HALF)])
    pltpu.sync_copy(oz, zo_out.at[s, pl.ds(col, _HALF)])


def _gather_sc(xt, yt, zt, idxT):
    mesh = plsc.VectorSubcoreMesh(core_axis_name="c", subcore_axis_name="s")
    return pl.kernel(
        _gather_sc_body,
        out_type=[
            jax.ShapeDtypeStruct((_B, _S), jnp.float32),
            jax.ShapeDtypeStruct((_B, _S), jnp.float32),
            jax.ShapeDtypeStruct((_B, _S), jnp.float32),
        ],
        mesh=mesh,
        compiler_params=pltpu.CompilerParams(needs_layout_passes=False),
        scratch_types=[
            pltpu.VMEM((_N,), jnp.float32),
            pltpu.VMEM((_N,), jnp.float32),
            pltpu.VMEM((_N,), jnp.float32),
            pltpu.VMEM((_HALF,), jnp.int32),
            pltpu.VMEM((_HALF,), jnp.float32),
            pltpu.VMEM((_HALF,), jnp.float32),
            pltpu.VMEM((_HALF,), jnp.float32),
            pltpu.SemaphoreType.DMA,
        ],
    )(xt, yt, zt, idxT)


def kernel(points):
    x = points[:, :, 0]
    y = points[:, :, 1]
    z = points[:, :, 2]
    idx = _fps_tc(x, y, z)
    xyz1_ind = idx.T
    xo, yo, zo = _gather_sc(x, y, z, xyz1_ind)
    xyz1 = jnp.stack([xo, yo, zo], axis=-1)
    return (xyz1_ind, xyz1)


# speedup vs baseline: 1.0998x; 1.0998x over previous
import functools

import jax
import jax.numpy as jnp
from jax import lax
from jax.experimental import pallas as pl
from jax.experimental.pallas import tpu as pltpu
from jax.experimental.pallas import tpu_sc as plsc

_B = 16
_N = 8192
_S = 2048
_HALF = _S // 2


_W = 512
_NB = _N // _W
_F = _W // 128


def _fold(t, op):
    acc = t[:, 0:128]
    for g in range(1, _F):
        acc = op(acc, t[:, g * 128:(g + 1) * 128])
    return acc


def _fps_tc_body(x_ref, y_ref, z_ref, idx_ref, dist_ref):
    r = jax.lax.broadcasted_iota(jnp.int32, (_B, _B), 0)
    c = jax.lax.broadcasted_iota(jnp.int32, (_B, _B), 1)
    eye = r == c
    dist_ref[...] = jnp.full((_B, _N), 1e10, jnp.float32)
    lane0 = jax.lax.broadcasted_iota(jnp.int32, (_B, _W), 1)

    def t_row(v):
        vb = jnp.broadcast_to(v, (_B, _B))
        return jnp.sum(jnp.where(eye, vb, jnp.zeros_like(vb)), axis=0,
                       keepdims=True)

    def body(i, far):
        idx_ref[pl.ds(i, 1), :] = t_row(far)

        ax = jnp.zeros((_B, 128), jnp.float32)
        ay = jnp.zeros((_B, 128), jnp.float32)
        az = jnp.zeros((_B, 128), jnp.float32)
        for blk in range(_NB):
            sl = pl.ds(blk * _W, _W)
            m2 = (lane0 + blk * _W) == far
            ax = ax + _fold(jnp.where(m2, x_ref[:, sl], 0.0), jnp.add)
            ay = ay + _fold(jnp.where(m2, y_ref[:, sl], 0.0), jnp.add)
            az = az + _fold(jnp.where(m2, z_ref[:, sl], 0.0), jnp.add)
        cx = jnp.sum(ax, axis=1, keepdims=True)
        cy = jnp.sum(ay, axis=1, keepdims=True)
        cz = jnp.sum(az, axis=1, keepdims=True)

        am = jnp.full((_B, 128), -1.0, jnp.float32)
        for blk in range(_NB):
            sl = pl.ds(blk * _W, _W)
            dx = x_ref[:, sl] - cx
            dy = y_ref[:, sl] - cy
            dz = z_ref[:, sl] - cz
            d = (dx * dx + dz * dz) + dy * dy
            nd = jnp.minimum(dist_ref[:, sl], d)
            dist_ref[:, sl] = nd
            am = jnp.maximum(am, _fold(nd, jnp.maximum))
        mx = jnp.max(am, axis=1, keepdims=True)

        ai = jnp.full((_B, 128), _N, jnp.int32)
        for blk in range(_NB):
            sl = pl.ds(blk * _W, _W)
            cand = jnp.where(dist_ref[:, sl] == mx, lane0 + blk * _W, _N)
            ai = jnp.minimum(ai, _fold(cand, jnp.minimum))
        far_new = jnp.min(ai, axis=1, keepdims=True)
        return far_new

    jax.lax.fori_loop(0, _S, body, jnp.zeros((_B, 1), jnp.int32), unroll=2)


def _fps_tc(x, y, z):
    return pl.pallas_call(
        _fps_tc_body,
        out_shape=jax.ShapeDtypeStruct((_S, _B), jnp.int32),
        scratch_shapes=[pltpu.VMEM((_B, _N), jnp.float32)],
    )(x, y, z)


_L = 16


def _gather_sc_body(xt, yt, zt, idxT, xo_out, yo_out, zo_out,
                    xs, ys, zs, idx_v, ox, oy, oz, sem):
    c = lax.axis_index("c")
    s = lax.axis_index("s")
    col = c * _HALF
    pltpu.sync_copy(xt.at[s], xs)
    pltpu.sync_copy(yt.at[s], ys)
    pltpu.sync_copy(zt.at[s], zs)
    pltpu.sync_copy(idxT.at[s, pl.ds(col, _HALF)], idx_v)

    @plsc.parallel_loop(0, _HALF, step=_L, unroll=8)
    def chunk(k):
        idxc = idx_v[pl.ds(k, _L)]
        ox[pl.ds(k, _L)] = plsc.load_gather(xs, [idxc])
        oy[pl.ds(k, _L)] = plsc.load_gather(ys, [idxc])
        oz[pl.ds(k, _L)] = plsc.load_gather(zs, [idxc])

    pltpu.sync_copy(ox, xo_out.at[s, pl.ds(col, _HALF)])
    pltpu.sync_copy(oy, yo_out.at[s, pl.ds(col, _HALF)])
    pltpu.sync_copy(oz, zo_out.at[s, pl.ds(col, _HALF)])


def _gather_sc(xt, yt, zt, idxT):
    mesh = plsc.VectorSubcoreMesh(core_axis_name="c", subcore_axis_name="s")
    return pl.kernel(
        _gather_sc_body,
        out_type=[
            jax.ShapeDtypeStruct((_B, _S), jnp.float32),
            jax.ShapeDtypeStruct((_B, _S), jnp.float32),
            jax.ShapeDtypeStruct((_B, _S), jnp.float32),
        ],
        mesh=mesh,
        compiler_params=pltpu.CompilerParams(needs_layout_passes=False),
        scratch_types=[
            pltpu.VMEM((_N,), jnp.float32),
            pltpu.VMEM((_N,), jnp.float32),
            pltpu.VMEM((_N,), jnp.float32),
            pltpu.VMEM((_HALF,), jnp.int32),
            pltpu.VMEM((_HALF,), jnp.float32),
            pltpu.VMEM((_HALF,), jnp.float32),
            pltpu.VMEM((_HALF,), jnp.float32),
            pltpu.SemaphoreType.DMA,
        ],
    )(xt, yt, zt, idxT)


def kernel(points):
    x = points[:, :, 0]
    y = points[:, :, 1]
    z = points[:, :, 2]
    idx = _fps_tc(x, y, z)
    xyz1_ind = idx.T
    xo, yo, zo = _gather_sc(x, y, z, xyz1_ind)
    xyz1 = jnp.stack([xo, yo, zo], axis=-1)
    return (xyz1_ind, xyz1)


# speedup vs baseline: 1.1514x; 1.0469x over previous
import functools

import jax
import jax.numpy as jnp
from jax import lax
from jax.experimental import pallas as pl
from jax.experimental.pallas import tpu as pltpu
from jax.experimental.pallas import tpu_sc as plsc

_B = 16
_N = 8192
_S = 2048
_HALF = _S // 2


_W = 512
_NB = _N // _W
_F = _W // 128


def _fold(t, op):
    acc = t[:, 0:128]
    for g in range(1, _F):
        acc = op(acc, t[:, g * 128:(g + 1) * 128])
    return acc


def _fps_tc_body(x_ref, y_ref, z_ref, idx_ref, dist_ref):
    r = jax.lax.broadcasted_iota(jnp.int32, (_B, _B), 0)
    c = jax.lax.broadcasted_iota(jnp.int32, (_B, _B), 1)
    eye = r == c
    dist_ref[...] = jnp.full((_B, _N), 1e10, jnp.float32)
    lane0 = jax.lax.broadcasted_iota(jnp.int32, (_B, _W), 1)

    def t_row(v):
        vb = jnp.broadcast_to(v, (_B, _B))
        return jnp.sum(jnp.where(eye, vb, jnp.zeros_like(vb)), axis=0,
                       keepdims=True)

    def body(i, far):
        idx_ref[pl.ds(i, 1), :] = t_row(far)

        ax = jnp.zeros((_B, 128), jnp.float32)
        ay = jnp.zeros((_B, 128), jnp.float32)
        az = jnp.zeros((_B, 128), jnp.float32)
        for blk in range(_NB):
            sl = pl.ds(blk * _W, _W)
            m2 = (lane0 + blk * _W) == far
            ax = ax + _fold(jnp.where(m2, x_ref[:, sl], 0.0), jnp.add)
            ay = ay + _fold(jnp.where(m2, y_ref[:, sl], 0.0), jnp.add)
            az = az + _fold(jnp.where(m2, z_ref[:, sl], 0.0), jnp.add)
        cx = jnp.sum(ax, axis=1, keepdims=True)
        cy = jnp.sum(ay, axis=1, keepdims=True)
        cz = jnp.sum(az, axis=1, keepdims=True)

        am = jnp.full((_B, 128), -1.0, jnp.float32)
        for blk in range(_NB):
            sl = pl.ds(blk * _W, _W)
            dx = x_ref[:, sl] - cx
            dy = y_ref[:, sl] - cy
            dz = z_ref[:, sl] - cz
            d = (dx * dx + dz * dz) + dy * dy
            nd = jnp.minimum(dist_ref[:, sl], d)
            dist_ref[:, sl] = nd
            am = jnp.maximum(am, _fold(nd, jnp.maximum))
        mx = jnp.max(am, axis=1, keepdims=True)

        ai = jnp.full((_B, 128), _N, jnp.int32)
        for blk in range(_NB):
            sl = pl.ds(blk * _W, _W)
            cand = jnp.where(dist_ref[:, sl] == mx, lane0 + blk * _W, _N)
            ai = jnp.minimum(ai, _fold(cand, jnp.minimum))
        far_new = jnp.min(ai, axis=1, keepdims=True)
        return far_new

    jax.lax.fori_loop(0, _S, body, jnp.zeros((_B, 1), jnp.int32), unroll=4)


def _fps_tc(x, y, z):
    return pl.pallas_call(
        _fps_tc_body,
        out_shape=jax.ShapeDtypeStruct((_S, _B), jnp.int32),
        scratch_shapes=[pltpu.VMEM((_B, _N), jnp.float32)],
    )(x, y, z)


_L = 16


def _gather_sc_body(xt, yt, zt, idxT, xo_out, yo_out, zo_out,
                    xs, ys, zs, idx_v, ox, oy, oz, sem):
    c = lax.axis_index("c")
    s = lax.axis_index("s")
    col = c * _HALF
    pltpu.sync_copy(xt.at[s], xs)
    pltpu.sync_copy(yt.at[s], ys)
    pltpu.sync_copy(zt.at[s], zs)
    pltpu.sync_copy(idxT.at[s, pl.ds(col, _HALF)], idx_v)

    @plsc.parallel_loop(0, _HALF, step=_L, unroll=8)
    def chunk(k):
        idxc = idx_v[pl.ds(k, _L)]
        ox[pl.ds(k, _L)] = plsc.load_gather(xs, [idxc])
        oy[pl.ds(k, _L)] = plsc.load_gather(ys, [idxc])
        oz[pl.ds(k, _L)] = plsc.load_gather(zs, [idxc])

    pltpu.sync_copy(ox, xo_out.at[s, pl.ds(col, _HALF)])
    pltpu.sync_copy(oy, yo_out.at[s, pl.ds(col, _HALF)])
    pltpu.sync_copy(oz, zo_out.at[s, pl.ds(col, _HALF)])


def _gather_sc(xt, yt, zt, idxT):
    mesh = plsc.VectorSubcoreMesh(core_axis_name="c", subcore_axis_name="s")
    return pl.kernel(
        _gather_sc_body,
        out_type=[
            jax.ShapeDtypeStruct((_B, _S), jnp.float32),
            jax.ShapeDtypeStruct((_B, _S), jnp.float32),
            jax.ShapeDtypeStruct((_B, _S), jnp.float32),
        ],
        mesh=mesh,
        compiler_params=pltpu.CompilerParams(needs_layout_passes=False),
        scratch_types=[
            pltpu.VMEM((_N,), jnp.float32),
            pltpu.VMEM((_N,), jnp.float32),
            pltpu.VMEM((_N,), jnp.float32),
            pltpu.VMEM((_HALF,), jnp.int32),
            pltpu.VMEM((_HALF,), jnp.float32),
            pltpu.VMEM((_HALF,), jnp.float32),
            pltpu.VMEM((_HALF,), jnp.float32),
            pltpu.SemaphoreType.DMA,
        ],
    )(xt, yt, zt, idxT)


def kernel(points):
    x = points[:, :, 0]
    y = points[:, :, 1]
    z = points[:, :, 2]
    idx = _fps_tc(x, y, z)
    xyz1_ind = idx.T
    xo, yo, zo = _gather_sc(x, y, z, xyz1_ind)
    xyz1 = jnp.stack([xo, yo, zo], axis=-1)
    return (xyz1_ind, xyz1)


# speedup vs baseline: 1.1729x; 1.0187x over previous
import functools

import jax
import jax.numpy as jnp
from jax import lax
from jax.experimental import pallas as pl
from jax.experimental.pallas import tpu as pltpu
from jax.experimental.pallas import tpu_sc as plsc

_B = 16
_N = 8192
_S = 2048
_HALF = _S // 2


_W = 512
_NB = _N // _W
_F = _W // 128


def _fold(t, op):
    acc = t[:, 0:128]
    for g in range(1, _F):
        acc = op(acc, t[:, g * 128:(g + 1) * 128])
    return acc


def _fps_tc_body(x_ref, y_ref, z_ref, idx_ref, dist_ref):
    r = jax.lax.broadcasted_iota(jnp.int32, (_B, _B), 0)
    c = jax.lax.broadcasted_iota(jnp.int32, (_B, _B), 1)
    eye = r == c
    dist_ref[...] = jnp.full((_B, _N), 1e10, jnp.float32)
    lane0 = jax.lax.broadcasted_iota(jnp.int32, (_B, _W), 1)

    def t_row(v):
        vb = jnp.broadcast_to(v, (_B, _B))
        return jnp.sum(jnp.where(eye, vb, jnp.zeros_like(vb)), axis=0,
                       keepdims=True)

    def body(i, far):
        idx_ref[pl.ds(i, 1), :] = t_row(far)

        ax = jnp.zeros((_B, 128), jnp.float32)
        ay = jnp.zeros((_B, 128), jnp.float32)
        az = jnp.zeros((_B, 128), jnp.float32)
        for blk in range(_NB):
            sl = pl.ds(blk * _W, _W)
            m2 = (lane0 + blk * _W) == far
            ax = ax + _fold(jnp.where(m2, x_ref[:, sl], 0.0), jnp.add)
            ay = ay + _fold(jnp.where(m2, y_ref[:, sl], 0.0), jnp.add)
            az = az + _fold(jnp.where(m2, z_ref[:, sl], 0.0), jnp.add)
        cx = jnp.sum(ax, axis=1, keepdims=True)
        cy = jnp.sum(ay, axis=1, keepdims=True)
        cz = jnp.sum(az, axis=1, keepdims=True)

        am = jnp.full((_B, 128), -1.0, jnp.float32)
        for blk in range(_NB):
            sl = pl.ds(blk * _W, _W)
            dx = x_ref[:, sl] - cx
            dy = y_ref[:, sl] - cy
            dz = z_ref[:, sl] - cz
            d = (dx * dx + dz * dz) + dy * dy
            nd = jnp.minimum(dist_ref[:, sl], d)
            dist_ref[:, sl] = nd
            am = jnp.maximum(am, _fold(nd, jnp.maximum))
        mx = jnp.max(am, axis=1, keepdims=True)

        ai = jnp.full((_B, 128), _N, jnp.int32)
        for blk in range(_NB):
            sl = pl.ds(blk * _W, _W)
            cand = jnp.where(dist_ref[:, sl] == mx, lane0 + blk * _W, _N)
            ai = jnp.minimum(ai, _fold(cand, jnp.minimum))
        far_new = jnp.min(ai, axis=1, keepdims=True)
        return far_new

    jax.lax.fori_loop(0, _S, body, jnp.zeros((_B, 1), jnp.int32), unroll=8)


def _fps_tc(x, y, z):
    return pl.pallas_call(
        _fps_tc_body,
        out_shape=jax.ShapeDtypeStruct((_S, _B), jnp.int32),
        scratch_shapes=[pltpu.VMEM((_B, _N), jnp.float32)],
    )(x, y, z)


_L = 16


def _gather_sc_body(xt, yt, zt, idxT, xo_out, yo_out, zo_out,
                    xs, ys, zs, idx_v, ox, oy, oz, sem):
    c = lax.axis_index("c")
    s = lax.axis_index("s")
    col = c * _HALF
    pltpu.sync_copy(xt.at[s], xs)
    pltpu.sync_copy(yt.at[s], ys)
    pltpu.sync_copy(zt.at[s], zs)
    pltpu.sync_copy(idxT.at[s, pl.ds(col, _HALF)], idx_v)

    @plsc.parallel_loop(0, _HALF, step=_L, unroll=8)
    def chunk(k):
        idxc = idx_v[pl.ds(k, _L)]
        ox[pl.ds(k, _L)] = plsc.load_gather(xs, [idxc])
        oy[pl.ds(k, _L)] = plsc.load_gather(ys, [idxc])
        oz[pl.ds(k, _L)] = plsc.load_gather(zs, [idxc])

    pltpu.sync_copy(ox, xo_out.at[s, pl.ds(col, _HALF)])
    pltpu.sync_copy(oy, yo_out.at[s, pl.ds(col, _HALF)])
    pltpu.sync_copy(oz, zo_out.at[s, pl.ds(col, _HALF)])


def _gather_sc(xt, yt, zt, idxT):
    mesh = plsc.VectorSubcoreMesh(core_axis_name="c", subcore_axis_name="s")
    return pl.kernel(
        _gather_sc_body,
        out_type=[
            jax.ShapeDtypeStruct((_B, _S), jnp.float32),
            jax.ShapeDtypeStruct((_B, _S), jnp.float32),
            jax.ShapeDtypeStruct((_B, _S), jnp.float32),
        ],
        mesh=mesh,
        compiler_params=pltpu.CompilerParams(needs_layout_passes=False),
        scratch_types=[
            pltpu.VMEM((_N,), jnp.float32),
            pltpu.VMEM((_N,), jnp.float32),
            pltpu.VMEM((_N,), jnp.float32),
            pltpu.VMEM((_HALF,), jnp.int32),
            pltpu.VMEM((_HALF,), jnp.float32),
            pltpu.VMEM((_HALF,), jnp.float32),
            pltpu.VMEM((_HALF,), jnp.float32),
            pltpu.SemaphoreType.DMA,
        ],
    )(xt, yt, zt, idxT)


def kernel(points):
    x = points[:, :, 0]
    y = points[:, :, 1]
    z = points[:, :, 2]
    idx = _fps_tc(x, y, z)
    xyz1_ind = idx.T
    xo, yo, zo = _gather_sc(x, y, z, xyz1_ind)
    xyz1 = jnp.stack([xo, yo, zo], axis=-1)
    return (xyz1_ind, xyz1)


# speedup vs baseline: 1.1920x; 1.0163x over previous
import functools

import jax
import jax.numpy as jnp
from jax import lax
from jax.experimental import pallas as pl
from jax.experimental.pallas import tpu as pltpu
from jax.experimental.pallas import tpu_sc as plsc

_B = 16
_N = 8192
_S = 2048
_HALF = _S // 2


_W = 512
_NB = _N // _W
_F = _W // 128


def _fold(t, op):
    acc = t[:, 0:128]
    for g in range(1, _F):
        acc = op(acc, t[:, g * 128:(g + 1) * 128])
    return acc


def _fps_tc_body(x_ref, y_ref, z_ref, idx_ref, dist_ref):
    r = jax.lax.broadcasted_iota(jnp.int32, (_B, _B), 0)
    c = jax.lax.broadcasted_iota(jnp.int32, (_B, _B), 1)
    eye = r == c
    dist_ref[...] = jnp.full((_B, _N), 1e10, jnp.float32)
    lane0 = jax.lax.broadcasted_iota(jnp.int32, (_B, _W), 1)

    def t_row(v):
        vb = jnp.broadcast_to(v, (_B, _B))
        return jnp.sum(jnp.where(eye, vb, jnp.zeros_like(vb)), axis=0,
                       keepdims=True)

    def body(i, far):
        idx_ref[pl.ds(i, 1), :] = t_row(far)

        ax = jnp.zeros((_B, 128), jnp.float32)
        ay = jnp.zeros((_B, 128), jnp.float32)
        az = jnp.zeros((_B, 128), jnp.float32)
        for blk in range(_NB):
            sl = pl.ds(blk * _W, _W)
            m2 = (lane0 + blk * _W) == far
            ax = ax + _fold(jnp.where(m2, x_ref[:, sl], 0.0), jnp.add)
            ay = ay + _fold(jnp.where(m2, y_ref[:, sl], 0.0), jnp.add)
            az = az + _fold(jnp.where(m2, z_ref[:, sl], 0.0), jnp.add)
        cx = jnp.sum(ax, axis=1, keepdims=True)
        cy = jnp.sum(ay, axis=1, keepdims=True)
        cz = jnp.sum(az, axis=1, keepdims=True)

        am = jnp.full((_B, 128), -1.0, jnp.float32)
        for blk in range(_NB):
            sl = pl.ds(blk * _W, _W)
            dx = x_ref[:, sl] - cx
            dy = y_ref[:, sl] - cy
            dz = z_ref[:, sl] - cz
            d = (dx * dx + dz * dz) + dy * dy
            nd = jnp.minimum(dist_ref[:, sl], d)
            dist_ref[:, sl] = nd
            am = jnp.maximum(am, _fold(nd, jnp.maximum))
        mx = jnp.max(am, axis=1, keepdims=True)

        ai = jnp.full((_B, 128), _N, jnp.int32)
        for blk in range(_NB):
            sl = pl.ds(blk * _W, _W)
            cand = jnp.where(dist_ref[:, sl] == mx, lane0 + blk * _W, _N)
            ai = jnp.minimum(ai, _fold(cand, jnp.minimum))
        far_new = jnp.min(ai, axis=1, keepdims=True)
        return far_new

    jax.lax.fori_loop(0, _S, body, jnp.zeros((_B, 1), jnp.int32), unroll=16)


def _fps_tc(x, y, z):
    return pl.pallas_call(
        _fps_tc_body,
        out_shape=jax.ShapeDtypeStruct((_S, _B), jnp.int32),
        scratch_shapes=[pltpu.VMEM((_B, _N), jnp.float32)],
    )(x, y, z)


_L = 16


def _gather_sc_body(xt, yt, zt, idxT, xo_out, yo_out, zo_out,
                    xs, ys, zs, idx_v, ox, oy, oz, sem):
    c = lax.axis_index("c")
    s = lax.axis_index("s")
    col = c * _HALF
    pltpu.sync_copy(xt.at[s], xs)
    pltpu.sync_copy(yt.at[s], ys)
    pltpu.sync_copy(zt.at[s], zs)
    pltpu.sync_copy(idxT.at[s, pl.ds(col, _HALF)], idx_v)

    @plsc.parallel_loop(0, _HALF, step=_L, unroll=16)
    def chunk(k):
        idxc = idx_v[pl.ds(k, _L)]
        ox[pl.ds(k, _L)] = plsc.load_gather(xs, [idxc])
        oy[pl.ds(k, _L)] = plsc.load_gather(ys, [idxc])
        oz[pl.ds(k, _L)] = plsc.load_gather(zs, [idxc])

    pltpu.sync_copy(ox, xo_out.at[s, pl.ds(col, _HALF)])
    pltpu.sync_copy(oy, yo_out.at[s, pl.ds(col, _HALF)])
    pltpu.sync_copy(oz, zo_out.at[s, pl.ds(col, _HALF)])


def _gather_sc(xt, yt, zt, idxT):
    mesh = plsc.VectorSubcoreMesh(core_axis_name="c", subcore_axis_name="s")
    return pl.kernel(
        _gather_sc_body,
        out_type=[
            jax.ShapeDtypeStruct((_B, _S), jnp.float32),
            jax.ShapeDtypeStruct((_B, _S), jnp.float32),
            jax.ShapeDtypeStruct((_B, _S), jnp.float32),
        ],
        mesh=mesh,
        compiler_params=pltpu.CompilerParams(needs_layout_passes=False),
        scratch_types=[
            pltpu.VMEM((_N,), jnp.float32),
            pltpu.VMEM((_N,), jnp.float32),
            pltpu.VMEM((_N,), jnp.float32),
            pltpu.VMEM((_HALF,), jnp.int32),
            pltpu.VMEM((_HALF,), jnp.float32),
            pltpu.VMEM((_HALF,), jnp.float32),
            pltpu.VMEM((_HALF,), jnp.float32),
            pltpu.SemaphoreType.DMA,
        ],
    )(xt, yt, zt, idxT)


def kernel(points):
    x = points[:, :, 0]
    y = points[:, :, 1]
    z = points[:, :, 2]
    idx = _fps_tc(x, y, z)
    xyz1_ind = idx.T
    xo, yo, zo = _gather_sc(x, y, z, xyz1_ind)
    xyz1 = jnp.stack([xo, yo, zo], axis=-1)
    return (xyz1_ind, xyz1)
